# baseline (speedup 1.0000x reference)
import jax
import jax.numpy as jnp
from jax.experimental import pallas as pl


def _score_body(x_ref, entT_ref, b_ref, o_ref):
    s = jnp.dot(x_ref[...], entT_ref[...], preferred_element_type=jnp.float32)
    o_ref[...] = jax.nn.sigmoid(s + b_ref[...])


def _score(x, entT, bias2d, block_n=1024):
    bs, h = x.shape
    n = entT.shape[1]
    return pl.pallas_call(
        _score_body,
        grid=(n // block_n,),
        in_specs=[
            pl.BlockSpec((bs, h), lambda i: (0, 0)),
            pl.BlockSpec((h, block_n), lambda i: (0, i)),
            pl.BlockSpec((1, block_n), lambda i: (0, i)),
        ],
        out_specs=pl.BlockSpec((bs, block_n), lambda i: (0, i)),
        out_shape=jax.ShapeDtypeStruct((bs, n), jnp.float32),
    )(x, entT, bias2d)


def _mlp3(x, W1, b1, W2, b2, W3, b3):
    x = jax.nn.relu(x @ W1 + b1)
    x = jax.nn.relu(x @ W2 + b2)
    return x @ W3 + b3


def _corr(x1, x2):
    x1 = x1 - jnp.mean(x1, axis=0, keepdims=True)
    x2 = x2 - jnp.mean(x2, axis=0, keepdims=True)
    s1 = jnp.sqrt(jnp.mean(x1 ** 2))
    s2 = jnp.sqrt(jnp.mean(x2 ** 2))
    return jnp.abs(jnp.mean(x1 * x2)) / (s1 * s2)


def _edge_layer(x, rel_emb, src, dst, rel_id, neigh_w, s_xx, n_ent, topk):
    e = src.shape[0]
    R = jax.lax.dot_general(x, rel_emb, (((1,), (1,)), ((), ())),
                            precision=jax.lax.Precision.HIGHEST)
    norm = s_xx + R[dst, rel_id]
    seg_max = jax.ops.segment_max(norm, dst, num_segments=n_ent)
    ex = jnp.exp(norm - seg_max[dst])
    seg_sum = jax.ops.segment_sum(ex, dst, num_segments=n_ent)
    attn = ex / seg_sum[dst]
    order = jnp.lexsort((-attn, dst))
    sd = dst[order]
    start = jnp.searchsorted(sd, sd, side='left')
    rank = jnp.arange(e) - start
    keep = jnp.zeros((e,), dtype=bool).at[order].set(rank < topk)
    w = jnp.where(keep, attn, 0.0)
    neigh = jax.ops.segment_sum(x[src] * w[:, None], dst, num_segments=n_ent)
    A = jnp.zeros((n_ent, rel_emb.shape[0]), jnp.float32).at[dst, rel_id].add(w)
    neigh = neigh + jnp.matmul(A, rel_emb, precision=jax.lax.Precision.HIGHEST)
    return jnp.tanh(neigh @ neigh_w)


def kernel(h_id, r_id, edge_index, rel_id, ent_emb, S_w, S_b, L_w, L_b,
           rel_emb_0, rel_emb_1, neigh_w_0, neigh_w_1, pred_rel_emb,
           conv_w, conv_b, fc_w, fc_b, ent_bias,
           phi_W1, phi_b1, phi_W2, phi_b2, phi_W3, phi_b3,
           psi_W1, psi_b1, psi_W2, psi_b2, psi_W3, psi_b3):
    n_ent, h = ent_emb.shape
    topk = 15
    src = edge_index[0]
    dst = edge_index[1]

    common = ent_emb @ S_w + S_b
    private = ent_emb @ L_w + L_b

    s_cc = jnp.sum(common[src] * common[dst], axis=1)
    s_pp = jnp.sum(private[src] * private[dst], axis=1)

    cs = []
    for rel_emb, neigh_w in ((rel_emb_0, neigh_w_0), (rel_emb_1, neigh_w_1)):
        for x, sxx in ((common, s_cc), (private, s_pp)):
            cs.append(_edge_layer(x, rel_emb, src, dst, rel_id, neigh_w, sxx,
                                  n_ent, topk))
    c1_0, c2_0, c1_1, c2_1 = cs
    ent = ent_emb + c1_0 + c2_0 + c1_1 + c2_1
    corr = _corr(
        _mlp3(c1_1, phi_W1, phi_b1, phi_W2, phi_b2, phi_W3, phi_b3),
        _mlp3(c2_1, psi_W1, psi_b1, psi_W2, psi_b2, psi_W3, psi_b3))

    bs = h_id.shape[0]
    kh, kw = 8, 16
    head = ent[h_id]
    rel = pred_rel_emb[r_id]
    img = jnp.concatenate([head.reshape(bs, 1, kh, kw),
                           rel.reshape(bs, 1, kh, kw)], axis=2)
    xconv = jax.lax.conv_general_dilated(
        img, conv_w, (1, 1), 'VALID', dimension_numbers=('NCHW', 'OIHW', 'NCHW'))
    xconv = jax.nn.relu(xconv + conv_b[None, :, None, None])
    xfc = jax.nn.relu(xconv.reshape(bs, -1) @ fc_w + fc_b)

    n_pad = ((n_ent + 1023) // 1024) * 1024
    entT = jnp.zeros((h, n_pad), jnp.float32).at[:, :n_ent].set(ent.T)
    bias2d = jnp.zeros((1, n_pad), jnp.float32).at[0, :n_ent].set(ent_bias)
    score = _score(xfc, entT, bias2d)[:, :n_ent]
    return (score, corr)


# speedup vs baseline: 1.0758x; 1.0758x over previous
import functools

import jax
import jax.numpy as jnp
from jax import lax
from jax.experimental import pallas as pl
from jax.experimental.pallas import tpu as pltpu
from jax.experimental.pallas import tpu_sc as plsc

_NW = 32
_C = 40


def _edge_norm_sc(common, private, rel0, rel1, src, dst, rid):
    e_total = src.shape[0]
    per_w = e_total // _NW
    n_chunks = per_w // _C
    mesh = plsc.VectorSubcoreMesh(core_axis_name="c", subcore_axis_name="s")

    @functools.partial(
        pl.kernel, mesh=mesh,
        out_type=[jax.ShapeDtypeStruct((e_total, 16), jnp.float32)] * 4,
        scratch_types=(
            [pltpu.VMEM((_C,), jnp.int32)] * 3
            + [pltpu.VMEM((_C, 128), jnp.float32)] * 6
            + [pltpu.VMEM((_C, 16), jnp.float32)] * 4
            + [pltpu.SemaphoreType.DMA] * 6
        ),
    )
    def k(common_h, private_h, rel0_h, rel1_h, src_h, dst_h, rid_h,
          o_c0, o_p0, o_c1, o_p1,
          srcv, dstv, ridv, csr, cdr, psr, pdr, r0r, r1r,
          b_c0, b_p0, b_c1, b_p1, s0, s1, s2, s3, s4, s5):
        wid = lax.axis_index("s") * 2 + lax.axis_index("c")
        base = wid * per_w

        def chunk(ci, carry):
            off = base + ci * _C
            pltpu.sync_copy(src_h.at[pl.ds(off, _C)], srcv)
            pltpu.sync_copy(dst_h.at[pl.ds(off, _C)], dstv)
            pltpu.sync_copy(rid_h.at[pl.ds(off, _C)], ridv)
            cps = [
                pltpu.async_copy(common_h.at[srcv], csr, s0),
                pltpu.async_copy(common_h.at[dstv], cdr, s1),
                pltpu.async_copy(private_h.at[srcv], psr, s2),
                pltpu.async_copy(private_h.at[dstv], pdr, s3),
                pltpu.async_copy(rel0_h.at[ridv], r0r, s4),
                pltpu.async_copy(rel1_h.at[ridv], r1r, s5),
            ]
            for cp in cps:
                cp.wait()

            def edge(i, c2):
                a_c0 = jnp.zeros((16,), jnp.float32)
                a_c1 = jnp.zeros((16,), jnp.float32)
                a_p0 = jnp.zeros((16,), jnp.float32)
                a_p1 = jnp.zeros((16,), jnp.float32)
                for s in range(8):
                    sl = pl.ds(s * 16, 16)
                    vcs = csr[i, sl]
                    vcd = cdr[i, sl]
                    vps = psr[i, sl]
                    vpd = pdr[i, sl]
                    v0 = r0r[i, sl]
                    v1 = r1r[i, sl]
                    a_c0 = a_c0 + (vcs + v0) * vcd
                    a_c1 = a_c1 + (vcs + v1) * vcd
                    a_p0 = a_p0 + (vps + v0) * vpd
                    a_p1 = a_p1 + (vps + v1) * vpd
                b_c0[i, :] = a_c0
                b_c1[i, :] = a_c1
                b_p0[i, :] = a_p0
                b_p1[i, :] = a_p1
                return c2

            lax.fori_loop(0, _C, edge, 0)
            pltpu.sync_copy(b_c0, o_c0.at[pl.ds(off, _C)])
            pltpu.sync_copy(b_p0, o_p0.at[pl.ds(off, _C)])
            pltpu.sync_copy(b_c1, o_c1.at[pl.ds(off, _C)])
            pltpu.sync_copy(b_p1, o_p1.at[pl.ds(off, _C)])
            return carry

        lax.fori_loop(0, n_chunks, chunk, 0)

    return k(common, private, rel0, rel1, src, dst, rid)


def _aggregate_sc(x_table, rel_l, src, dst, rid, w, zeros_n):
    n = x_table.shape[0]
    e_total = src.shape[0]
    per_w = e_total // _NW
    n_chunks = per_w // _C
    dump_rows = 200
    n_dump = n // dump_rows
    mesh = plsc.VectorSubcoreMesh(core_axis_name="c", subcore_axis_name="s")

    @functools.partial(
        pl.kernel, mesh=mesh,
        out_type=[jax.ShapeDtypeStruct((n, 128), jnp.float32)] * 2,
        scratch_types=(
            [pltpu.VMEM((_C,), jnp.int32)] * 3
            + [pltpu.VMEM((_C,), jnp.float32)]
            + [pltpu.VMEM((_C, 128), jnp.float32)] * 3
            + [pltpu.VMEM_SHARED((n, 128), jnp.float32)]
            + [pltpu.VMEM((dump_rows, 128), jnp.float32)]
            + [pltpu.SemaphoreType.DMA] * 2
        ),
    )
    def k(x_h, rel_h, src_h, dst_h, rid_h, w_h, zeros_h, out0, out1,
          srcv, dstv, ridv, wv, xr, rr, comb, accum, dump, sa, sb):
        cc = lax.axis_index("c")
        sid = lax.axis_index("s")
        wid = sid * 2 + cc
        base = wid * per_w

        @pl.when(sid == 0)
        def _init():
            pltpu.sync_copy(zeros_h, accum)

        plsc.subcore_barrier()

        def chunk(ci, carry):
            off = base + ci * _C
            pltpu.sync_copy(src_h.at[pl.ds(off, _C)], srcv)
            pltpu.sync_copy(dst_h.at[pl.ds(off, _C)], dstv)
            pltpu.sync_copy(rid_h.at[pl.ds(off, _C)], ridv)
            pltpu.sync_copy(w_h.at[pl.ds(off, _C)], wv)
            cp0 = pltpu.async_copy(x_h.at[srcv], xr, sa)
            cp1 = pltpu.async_copy(rel_h.at[ridv], rr, sb)
            cp0.wait()
            cp1.wait()

            for g0 in range(0, _C, 16):
                gs = min(g0, _C - 16)
                wvec = wv[pl.ds(gs, 16)]
                for i in range(g0, min(g0 + 16, _C)):
                    w_s = wvec[i - gs]
                    for s in range(8):
                        sl = pl.ds(s * 16, 16)
                        comb[i, sl] = (xr[i, sl] + rr[i, sl]) * w_s
            pltpu.sync_copy(comb, accum.at[dstv], add=True)
            return carry

        lax.fori_loop(0, n_chunks, chunk, 0)
        plsc.subcore_barrier()

        for t in range((n_dump + 15) // 16):
            ch = sid + t * 16

            @pl.when(ch < n_dump)
            def _dump():
                r0 = ch * dump_rows
                pltpu.sync_copy(accum.at[pl.ds(r0, dump_rows)], dump)

                @pl.when(cc == 0)
                def _d0():
                    pltpu.sync_copy(dump, out0.at[pl.ds(r0, dump_rows)])

                @pl.when(cc == 1)
                def _d1():
                    pltpu.sync_copy(dump, out1.at[pl.ds(r0, dump_rows)])

    p0, p1 = k(x_table, rel_l, src, dst, rid, w, zeros_n)
    return p0 + p1


def _score_body(x_ref, entT_ref, b_ref, o_ref):
    s = jnp.dot(x_ref[...], entT_ref[...], preferred_element_type=jnp.float32)
    o_ref[...] = jax.nn.sigmoid(s + b_ref[...])


def _score(x, entT, bias2d, block_n=1024):
    bs, h = x.shape
    n = entT.shape[1]
    return pl.pallas_call(
        _score_body,
        grid=(n // block_n,),
        in_specs=[
            pl.BlockSpec((bs, h), lambda i: (0, 0)),
            pl.BlockSpec((h, block_n), lambda i: (0, i)),
            pl.BlockSpec((1, block_n), lambda i: (0, i)),
        ],
        out_specs=pl.BlockSpec((bs, block_n), lambda i: (0, i)),
        out_shape=jax.ShapeDtypeStruct((bs, n), jnp.float32),
    )(x, entT, bias2d)


def _mlp3(x, W1, b1, W2, b2, W3, b3):
    x = jax.nn.relu(x @ W1 + b1)
    x = jax.nn.relu(x @ W2 + b2)
    return x @ W3 + b3


def _corr(x1, x2):
    x1 = x1 - jnp.mean(x1, axis=0, keepdims=True)
    x2 = x2 - jnp.mean(x2, axis=0, keepdims=True)
    s1 = jnp.sqrt(jnp.mean(x1 ** 2))
    s2 = jnp.sqrt(jnp.mean(x2 ** 2))
    return jnp.abs(jnp.mean(x1 * x2)) / (s1 * s2)


def _softmax_topk_w(norm, dst, n_ent, topk):
    e = norm.shape[0]
    seg_max = jax.ops.segment_max(norm, dst, num_segments=n_ent)
    ex = jnp.exp(norm - seg_max[dst])
    seg_sum = jax.ops.segment_sum(ex, dst, num_segments=n_ent)
    attn = ex / seg_sum[dst]
    order = jnp.lexsort((-attn, dst))
    sd = dst[order]
    start = jnp.searchsorted(sd, sd, side='left')
    rank = jnp.arange(e) - start
    keep = jnp.zeros((e,), dtype=bool).at[order].set(rank < topk)
    return jnp.where(keep, attn, 0.0)


def kernel(h_id, r_id, edge_index, rel_id, ent_emb, S_w, S_b, L_w, L_b,
           rel_emb_0, rel_emb_1, neigh_w_0, neigh_w_1, pred_rel_emb,
           conv_w, conv_b, fc_w, fc_b, ent_bias,
           phi_W1, phi_b1, phi_W2, phi_b2, phi_W3, phi_b3,
           psi_W1, psi_b1, psi_W2, psi_b2, psi_W3, psi_b3):
    n_ent, h = ent_emb.shape
    topk = 15
    src = edge_index[0]
    dst = edge_index[1]

    common = ent_emb @ S_w + S_b
    private = ent_emb @ L_w + L_b

    pc0, pp0, pc1, pp1 = _edge_norm_sc(common, private, rel_emb_0, rel_emb_1,
                                       src, dst, rel_id)
    norms = {
        ('c', 0): jnp.sum(pc0, axis=1),
        ('p', 0): jnp.sum(pp0, axis=1),
        ('c', 1): jnp.sum(pc1, axis=1),
        ('p', 1): jnp.sum(pp1, axis=1),
    }

    zeros_n = jnp.zeros((n_ent, 128), jnp.float32)
    tables = {'c': common, 'p': private}
    rels = {0: rel_emb_0, 1: rel_emb_1}
    nw = {0: neigh_w_0, 1: neigh_w_1}
    cs = {}
    for l in (0, 1):
        for xk in ('c', 'p'):
            w = _softmax_topk_w(norms[(xk, l)], dst, n_ent, topk)
            neigh = _aggregate_sc(tables[xk], rels[l], src, dst, rel_id, w,
                                  zeros_n)
            cs[(xk, l)] = jnp.tanh(neigh @ nw[l])

    ent = ent_emb + cs[('c', 0)] + cs[('p', 0)] + cs[('c', 1)] + cs[('p', 1)]
    corr = _corr(
        _mlp3(cs[('c', 1)], phi_W1, phi_b1, phi_W2, phi_b2, phi_W3, phi_b3),
        _mlp3(cs[('p', 1)], psi_W1, psi_b1, psi_W2, psi_b2, psi_W3, psi_b3))

    bs = h_id.shape[0]
    kh, kw = 8, 16
    head = ent[h_id]
    rel = pred_rel_emb[r_id]
    img = jnp.concatenate([head.reshape(bs, 1, kh, kw),
                           rel.reshape(bs, 1, kh, kw)], axis=2)
    xconv = jax.lax.conv_general_dilated(
        img, conv_w, (1, 1), 'VALID', dimension_numbers=('NCHW', 'OIHW', 'NCHW'))
    xconv = jax.nn.relu(xconv + conv_b[None, :, None, None])
    xfc = jax.nn.relu(xconv.reshape(bs, -1) @ fc_w + fc_b)

    n_pad = ((n_ent + 1023) // 1024) * 1024
    entT = jnp.zeros((h, n_pad), jnp.float32).at[:, :n_ent].set(ent.T)
    bias2d = jnp.zeros((1, n_pad), jnp.float32).at[0, :n_ent].set(ent_bias)
    score = _score(xfc, entT, bias2d)[:, :n_ent]
    return (score, corr)


# speedup vs baseline: 4.2185x; 3.9214x over previous
import functools

import jax
import jax.numpy as jnp
from jax import lax
from jax.experimental import pallas as pl
from jax.experimental.pallas import tpu as pltpu
from jax.experimental.pallas import tpu_sc as plsc

_NW = 32
_C = 88
_CA = 120


def _edge_norm_sc(common, private, rel0, rel1, src, dst, rid):
    e_total = src.shape[0]
    per_w = e_total // _NW
    n_chunks = -(-per_w // _C)
    mesh = plsc.VectorSubcoreMesh(core_axis_name="c", subcore_axis_name="s")

    @functools.partial(
        pl.kernel, mesh=mesh,
        out_type=[jax.ShapeDtypeStruct((e_total, 16), jnp.float32)] * 4,
        scratch_types=(
            [pltpu.VMEM((_C,), jnp.int32)] * 3
            + [pltpu.VMEM((_C, 128), jnp.float32)] * 6
            + [pltpu.VMEM((_C, 16), jnp.float32)] * 4
            + [pltpu.SemaphoreType.DMA] * 6
        ),
    )
    def k(common_h, private_h, rel0_h, rel1_h, src_h, dst_h, rid_h,
          o_c0, o_p0, o_c1, o_p1,
          srcv, dstv, ridv, csr, cdr, psr, pdr, r0r, r1r,
          b_c0, b_p0, b_c1, b_p1, s0, s1, s2, s3, s4, s5):
        wid = lax.axis_index("s") * 2 + lax.axis_index("c")
        base = wid * per_w

        def chunk(ci, carry):
            off = base + jnp.minimum(ci * _C, per_w - _C)
            off = pl.multiple_of(off, 8)
            pltpu.sync_copy(src_h.at[pl.ds(off, _C)], srcv)
            pltpu.sync_copy(dst_h.at[pl.ds(off, _C)], dstv)
            pltpu.sync_copy(rid_h.at[pl.ds(off, _C)], ridv)
            cps = [
                pltpu.async_copy(common_h.at[srcv], csr, s0),
                pltpu.async_copy(common_h.at[dstv], cdr, s1),
                pltpu.async_copy(private_h.at[srcv], psr, s2),
                pltpu.async_copy(private_h.at[dstv], pdr, s3),
                pltpu.async_copy(rel0_h.at[ridv], r0r, s4),
                pltpu.async_copy(rel1_h.at[ridv], r1r, s5),
            ]
            for cp in cps:
                cp.wait()

            def edge(i, c2):
                a_c0 = jnp.zeros((16,), jnp.float32)
                a_c1 = jnp.zeros((16,), jnp.float32)
                a_p0 = jnp.zeros((16,), jnp.float32)
                a_p1 = jnp.zeros((16,), jnp.float32)
                for s in range(8):
                    sl = pl.ds(s * 16, 16)
                    vcs = csr[i, sl]
                    vcd = cdr[i, sl]
                    vps = psr[i, sl]
                    vpd = pdr[i, sl]
                    v0 = r0r[i, sl]
                    v1 = r1r[i, sl]
                    a_c0 = a_c0 + (vcs + v0) * vcd
                    a_c1 = a_c1 + (vcs + v1) * vcd
                    a_p0 = a_p0 + (vps + v0) * vpd
                    a_p1 = a_p1 + (vps + v1) * vpd
                b_c0[i, :] = a_c0
                b_c1[i, :] = a_c1
                b_p0[i, :] = a_p0
                b_p1[i, :] = a_p1
                return c2

            lax.fori_loop(0, _C, edge, 0)
            pltpu.sync_copy(b_c0, o_c0.at[pl.ds(off, _C)])
            pltpu.sync_copy(b_p0, o_p0.at[pl.ds(off, _C)])
            pltpu.sync_copy(b_c1, o_c1.at[pl.ds(off, _C)])
            pltpu.sync_copy(b_p1, o_p1.at[pl.ds(off, _C)])
            return carry

        lax.fori_loop(0, n_chunks, chunk, 0)

    return k(common, private, rel0, rel1, src, dst, rid)


def _aggregate_sc(x_table, rel_l, src, dst, rid, w, zeros_n):
    n = x_table.shape[0]
    e_total = src.shape[0]
    per_w = e_total // _NW
    n_chunks = -(-per_w // _CA)
    dump_rows = 40
    n_dump = n // dump_rows
    mesh = plsc.VectorSubcoreMesh(core_axis_name="c", subcore_axis_name="s")

    @functools.partial(
        pl.kernel, mesh=mesh,
        out_type=[jax.ShapeDtypeStruct((n, 128), jnp.float32)] * 2,
        scratch_types=(
            [pltpu.VMEM((_CA,), jnp.int32)] * 3
            + [pltpu.VMEM((_CA,), jnp.float32)]
            + [pltpu.VMEM((_CA, 128), jnp.float32)] * 2
            + [pltpu.VMEM_SHARED((n, 128), jnp.float32)]
            + [pltpu.VMEM((dump_rows, 128), jnp.float32)]
            + [pltpu.SemaphoreType.DMA] * 2
        ),
    )
    def k(x_h, rel_h, src_h, dst_h, rid_h, w_h, zeros_h, out0, out1,
          srcv, dstv, ridv, wv, xr, rr, accum, dump, sa, sb):
        cc = lax.axis_index("c")
        sid = lax.axis_index("s")
        wid = sid * 2 + cc
        base = wid * per_w

        @pl.when(sid == 0)
        def _init():
            pltpu.sync_copy(zeros_h, accum)

        plsc.subcore_barrier()

        overlap = n_chunks * _CA - per_w

        def chunk(ci, carry):
            off = base + jnp.minimum(ci * _CA, per_w - _CA)
            off = pl.multiple_of(off, 8)
            dup_zero = jnp.where(ci == n_chunks - 1, 0.0, 1.0)
            pltpu.sync_copy(src_h.at[pl.ds(off, _CA)], srcv)
            pltpu.sync_copy(dst_h.at[pl.ds(off, _CA)], dstv)
            pltpu.sync_copy(rid_h.at[pl.ds(off, _CA)], ridv)
            pltpu.sync_copy(w_h.at[pl.ds(off, _CA)], wv)
            cp0 = pltpu.async_copy(x_h.at[srcv], xr, sa)
            cp1 = pltpu.async_copy(rel_h.at[ridv], rr, sb)
            cp0.wait()
            cp1.wait()

            for g0 in range(0, _CA, 16):
                gs = min(g0, _CA - 16)
                wvec = wv[pl.ds(gs, 16)]
                for i in range(g0, min(g0 + 16, _CA)):
                    w_s = wvec[i - gs]
                    if i < overlap:
                        w_s = w_s * dup_zero
                    for s in range(8):
                        sl = pl.ds(s * 16, 16)
                        xr[i, sl] = (xr[i, sl] + rr[i, sl]) * w_s
            pltpu.sync_copy(xr, accum.at[dstv], add=True)
            return carry

        lax.fori_loop(0, n_chunks, chunk, 0)
        plsc.subcore_barrier()

        for t in range((n_dump + 15) // 16):
            ch = sid + t * 16

            @pl.when(ch < n_dump)
            def _dump():
                r0 = ch * dump_rows
                pltpu.sync_copy(accum.at[pl.ds(r0, dump_rows)], dump)

                @pl.when(cc == 0)
                def _d0():
                    pltpu.sync_copy(dump, out0.at[pl.ds(r0, dump_rows)])

                @pl.when(cc == 1)
                def _d1():
                    pltpu.sync_copy(dump, out1.at[pl.ds(r0, dump_rows)])

    p0, p1 = k(x_table, rel_l, src, dst, rid, w, zeros_n)
    return p0 + p1


def _score_body(x_ref, entT_ref, b_ref, o_ref):
    s = jnp.dot(x_ref[...], entT_ref[...], preferred_element_type=jnp.float32)
    o_ref[...] = jax.nn.sigmoid(s + b_ref[...])


def _score(x, entT, bias2d, block_n=1024):
    bs, h = x.shape
    n = entT.shape[1]
    return pl.pallas_call(
        _score_body,
        grid=(n // block_n,),
        in_specs=[
            pl.BlockSpec((bs, h), lambda i: (0, 0)),
            pl.BlockSpec((h, block_n), lambda i: (0, i)),
            pl.BlockSpec((1, block_n), lambda i: (0, i)),
        ],
        out_specs=pl.BlockSpec((bs, block_n), lambda i: (0, i)),
        out_shape=jax.ShapeDtypeStruct((bs, n), jnp.float32),
    )(x, entT, bias2d)


def _mlp3(x, W1, b1, W2, b2, W3, b3):
    x = jax.nn.relu(x @ W1 + b1)
    x = jax.nn.relu(x @ W2 + b2)
    return x @ W3 + b3


def _corr(x1, x2):
    x1 = x1 - jnp.mean(x1, axis=0, keepdims=True)
    x2 = x2 - jnp.mean(x2, axis=0, keepdims=True)
    s1 = jnp.sqrt(jnp.mean(x1 ** 2))
    s2 = jnp.sqrt(jnp.mean(x2 ** 2))
    return jnp.abs(jnp.mean(x1 * x2)) / (s1 * s2)


def _seg_scan(flags_start, vals, op):
    def comb(a, b):
        af, av = a
        bf, bv = b
        return (jnp.maximum(af, bf), jnp.where(bf > 0, bv, op(av, bv)))
    _, out = jax.lax.associative_scan(
        comb, (flags_start.astype(jnp.float32), vals))
    return out


def _softmax_topk_sorted(norm, dst, src, rid, topk):
    e = norm.shape[0]
    sd, neg_s, src_s, rid_s = jax.lax.sort(
        (dst, -norm, src, rid), num_keys=2, is_stable=True)
    nrm_s = -neg_s
    idx = jnp.arange(e)
    is_start = jnp.concatenate(
        [jnp.ones((1,), bool), sd[1:] != sd[:-1]])
    is_end = jnp.concatenate(
        [sd[1:] != sd[:-1], jnp.ones((1,), bool)])
    start_pos = jax.lax.associative_scan(
        jnp.maximum, jnp.where(is_start, idx, -1))
    rank = idx - start_pos
    ex = jnp.exp(nrm_s)
    cums = _seg_scan(is_start, ex, jnp.add)
    seg_total = jnp.flip(
        _seg_scan(jnp.flip(is_end), jnp.flip(cums), jnp.maximum))
    w_s = jnp.where(rank < topk, ex / seg_total, 0.0)
    return sd, src_s, rid_s, w_s


def kernel(h_id, r_id, edge_index, rel_id, ent_emb, S_w, S_b, L_w, L_b,
           rel_emb_0, rel_emb_1, neigh_w_0, neigh_w_1, pred_rel_emb,
           conv_w, conv_b, fc_w, fc_b, ent_bias,
           phi_W1, phi_b1, phi_W2, phi_b2, phi_W3, phi_b3,
           psi_W1, psi_b1, psi_W2, psi_b2, psi_W3, psi_b3):
    n_ent, h = ent_emb.shape
    topk = 15
    src = edge_index[0]
    dst = edge_index[1]

    common = ent_emb @ S_w + S_b
    private = ent_emb @ L_w + L_b

    pc0, pp0, pc1, pp1 = _edge_norm_sc(common, private, rel_emb_0, rel_emb_1,
                                       src, dst, rel_id)
    norms = {
        ('c', 0): jnp.sum(pc0, axis=1),
        ('p', 0): jnp.sum(pp0, axis=1),
        ('c', 1): jnp.sum(pc1, axis=1),
        ('p', 1): jnp.sum(pp1, axis=1),
    }

    zeros_n = jnp.zeros((n_ent, 128), jnp.float32)
    tables = {'c': common, 'p': private}
    rels = {0: rel_emb_0, 1: rel_emb_1}
    nw = {0: neigh_w_0, 1: neigh_w_1}
    cs = {}
    for l in (0, 1):
        for xk in ('c', 'p'):
            sd, src_s, rid_s, w_s = _softmax_topk_sorted(
                norms[(xk, l)], dst, src, rel_id, topk)
            neigh = _aggregate_sc(tables[xk], rels[l], src_s, sd, rid_s, w_s,
                                  zeros_n)
            cs[(xk, l)] = jnp.tanh(neigh @ nw[l])

    ent = ent_emb + cs[('c', 0)] + cs[('p', 0)] + cs[('c', 1)] + cs[('p', 1)]
    corr = _corr(
        _mlp3(cs[('c', 1)], phi_W1, phi_b1, phi_W2, phi_b2, phi_W3, phi_b3),
        _mlp3(cs[('p', 1)], psi_W1, psi_b1, psi_W2, psi_b2, psi_W3, psi_b3))

    bs = h_id.shape[0]
    kh, kw = 8, 16
    head = ent[h_id]
    rel = pred_rel_emb[r_id]
    img = jnp.concatenate([head.reshape(bs, 1, kh, kw),
                           rel.reshape(bs, 1, kh, kw)], axis=2)
    xconv = jax.lax.conv_general_dilated(
        img, conv_w, (1, 1), 'VALID', dimension_numbers=('NCHW', 'OIHW', 'NCHW'))
    xconv = jax.nn.relu(xconv + conv_b[None, :, None, None])
    xfc = jax.nn.relu(xconv.reshape(bs, -1) @ fc_w + fc_b)

    n_pad = ((n_ent + 1023) // 1024) * 1024
    entT = jnp.zeros((h, n_pad), jnp.float32).at[:, :n_ent].set(ent.T)
    bias2d = jnp.zeros((1, n_pad), jnp.float32).at[0, :n_ent].set(ent_bias)
    score = _score(xfc, entT, bias2d)[:, :n_ent]
    return (score, corr)


# speedup vs baseline: 4.2837x; 1.0155x over previous
import functools

import jax
import jax.numpy as jnp
from jax import lax
from jax.experimental import pallas as pl
from jax.experimental.pallas import tpu as pltpu
from jax.experimental.pallas import tpu_sc as plsc

_NW = 32
_C = 88
_CA = 120


def _edge_norm_sc(common, private, rel0, rel1, src, dst, rid):
    e_total = src.shape[0]
    per_w = e_total // _NW
    n_chunks = -(-per_w // _C)
    mesh = plsc.VectorSubcoreMesh(core_axis_name="c", subcore_axis_name="s")

    @functools.partial(
        pl.kernel, mesh=mesh,
        out_type=[jax.ShapeDtypeStruct((e_total, 16), jnp.float32)] * 4,
        scratch_types=(
            [pltpu.VMEM((_C,), jnp.int32)] * 3
            + [pltpu.VMEM((_C, 128), jnp.float32)] * 6
            + [pltpu.VMEM((_C, 16), jnp.float32)] * 4
            + [pltpu.SemaphoreType.DMA] * 9
        ),
    )
    def k(common_h, private_h, rel0_h, rel1_h, src_h, dst_h, rid_h,
          o_c0, o_p0, o_c1, o_p1,
          srcv, dstv, ridv, csr, cdr, psr, pdr, r0r, r1r,
          b_c0, b_p0, b_c1, b_p1, s0, s1, s2, s3, s4, s5, s6, s7, s8):
        wid = lax.axis_index("s") * 2 + lax.axis_index("c")
        base = wid * per_w

        def chunk(ci, carry):
            off = base + jnp.minimum(ci * _C, per_w - _C)
            off = pl.multiple_of(off, 8)
            idx_cps = [
                pltpu.async_copy(src_h.at[pl.ds(off, _C)], srcv, s6),
                pltpu.async_copy(dst_h.at[pl.ds(off, _C)], dstv, s7),
                pltpu.async_copy(rid_h.at[pl.ds(off, _C)], ridv, s8),
            ]
            for cp in idx_cps:
                cp.wait()
            cps = [
                pltpu.async_copy(common_h.at[srcv], csr, s0),
                pltpu.async_copy(common_h.at[dstv], cdr, s1),
                pltpu.async_copy(private_h.at[srcv], psr, s2),
                pltpu.async_copy(private_h.at[dstv], pdr, s3),
                pltpu.async_copy(rel0_h.at[ridv], r0r, s4),
                pltpu.async_copy(rel1_h.at[ridv], r1r, s5),
            ]
            for cp in cps:
                cp.wait()

            def edge(i, c2):
                a_c0 = jnp.zeros((16,), jnp.float32)
                a_c1 = jnp.zeros((16,), jnp.float32)
                a_p0 = jnp.zeros((16,), jnp.float32)
                a_p1 = jnp.zeros((16,), jnp.float32)
                for s in range(8):
                    sl = pl.ds(s * 16, 16)
                    vcs = csr[i, sl]
                    vcd = cdr[i, sl]
                    vps = psr[i, sl]
                    vpd = pdr[i, sl]
                    v0 = r0r[i, sl]
                    v1 = r1r[i, sl]
                    a_c0 = a_c0 + (vcs + v0) * vcd
                    a_c1 = a_c1 + (vcs + v1) * vcd
                    a_p0 = a_p0 + (vps + v0) * vpd
                    a_p1 = a_p1 + (vps + v1) * vpd
                b_c0[i, :] = a_c0
                b_c1[i, :] = a_c1
                b_p0[i, :] = a_p0
                b_p1[i, :] = a_p1
                return c2

            lax.fori_loop(0, _C, edge, 0)
            out_cps = [
                pltpu.async_copy(b_c0, o_c0.at[pl.ds(off, _C)], s0),
                pltpu.async_copy(b_p0, o_p0.at[pl.ds(off, _C)], s1),
                pltpu.async_copy(b_c1, o_c1.at[pl.ds(off, _C)], s2),
                pltpu.async_copy(b_p1, o_p1.at[pl.ds(off, _C)], s3),
            ]
            for cp in out_cps:
                cp.wait()
            return carry

        lax.fori_loop(0, n_chunks, chunk, 0)

    return k(common, private, rel0, rel1, src, dst, rid)


def _aggregate_sc(x_table, rel_l, src, dst, rid, w, zeros_n):
    n = x_table.shape[0]
    e_total = src.shape[0]
    per_w = e_total // _NW
    n_chunks = -(-per_w // _CA)
    dump_rows = 40
    n_dump = n // dump_rows
    mesh = plsc.VectorSubcoreMesh(core_axis_name="c", subcore_axis_name="s")

    @functools.partial(
        pl.kernel, mesh=mesh,
        out_type=[jax.ShapeDtypeStruct((n, 128), jnp.float32)] * 2,
        scratch_types=(
            [pltpu.VMEM((_CA,), jnp.int32)] * 3
            + [pltpu.VMEM((_CA,), jnp.float32)]
            + [pltpu.VMEM((_CA, 128), jnp.float32)] * 2
            + [pltpu.VMEM_SHARED((n, 128), jnp.float32)]
            + [pltpu.VMEM((dump_rows, 128), jnp.float32)]
            + [pltpu.SemaphoreType.DMA] * 6
        ),
    )
    def k(x_h, rel_h, src_h, dst_h, rid_h, w_h, zeros_h, out0, out1,
          srcv, dstv, ridv, wv, xr, rr, accum, dump, sa, sb,
          si0, si1, si2, si3):
        cc = lax.axis_index("c")
        sid = lax.axis_index("s")
        wid = sid * 2 + cc
        base = wid * per_w

        @pl.when(sid == 0)
        def _init():
            pltpu.sync_copy(zeros_h, accum)

        plsc.subcore_barrier()

        overlap = n_chunks * _CA - per_w

        def chunk(ci, carry):
            off = base + jnp.minimum(ci * _CA, per_w - _CA)
            off = pl.multiple_of(off, 8)
            dup_zero = jnp.where(ci == n_chunks - 1, 0.0, 1.0)
            idx_cps = [
                pltpu.async_copy(src_h.at[pl.ds(off, _CA)], srcv, si0),
                pltpu.async_copy(dst_h.at[pl.ds(off, _CA)], dstv, si1),
                pltpu.async_copy(rid_h.at[pl.ds(off, _CA)], ridv, si2),
                pltpu.async_copy(w_h.at[pl.ds(off, _CA)], wv, si3),
            ]
            for cp in idx_cps:
                cp.wait()
            cp0 = pltpu.async_copy(x_h.at[srcv], xr, sa)
            cp1 = pltpu.async_copy(rel_h.at[ridv], rr, sb)
            cp0.wait()
            cp1.wait()

            for g0 in range(0, _CA, 16):
                gs = min(g0, _CA - 16)
                wvec = wv[pl.ds(gs, 16)]
                for i in range(g0, min(g0 + 16, _CA)):
                    w_s = wvec[i - gs]
                    if i < overlap:
                        w_s = w_s * dup_zero
                    for s in range(8):
                        sl = pl.ds(s * 16, 16)
                        xr[i, sl] = (xr[i, sl] + rr[i, sl]) * w_s
            pltpu.sync_copy(xr, accum.at[dstv], add=True)
            return carry

        lax.fori_loop(0, n_chunks, chunk, 0)
        plsc.subcore_barrier()

        for t in range((n_dump + 15) // 16):
            ch = sid + t * 16

            @pl.when(ch < n_dump)
            def _dump():
                r0 = ch * dump_rows
                pltpu.sync_copy(accum.at[pl.ds(r0, dump_rows)], dump)

                @pl.when(cc == 0)
                def _d0():
                    pltpu.sync_copy(dump, out0.at[pl.ds(r0, dump_rows)])

                @pl.when(cc == 1)
                def _d1():
                    pltpu.sync_copy(dump, out1.at[pl.ds(r0, dump_rows)])

    p0, p1 = k(x_table, rel_l, src, dst, rid, w, zeros_n)
    return p0 + p1


def _score_body(x_ref, entT_ref, b_ref, o_ref):
    s = jnp.dot(x_ref[...], entT_ref[...], preferred_element_type=jnp.float32)
    o_ref[...] = jax.nn.sigmoid(s + b_ref[...])


def _score(x, entT, bias2d, block_n=1024):
    bs, h = x.shape
    n = entT.shape[1]
    return pl.pallas_call(
        _score_body,
        grid=(n // block_n,),
        in_specs=[
            pl.BlockSpec((bs, h), lambda i: (0, 0)),
            pl.BlockSpec((h, block_n), lambda i: (0, i)),
            pl.BlockSpec((1, block_n), lambda i: (0, i)),
        ],
        out_specs=pl.BlockSpec((bs, block_n), lambda i: (0, i)),
        out_shape=jax.ShapeDtypeStruct((bs, n), jnp.float32),
    )(x, entT, bias2d)


def _mlp3(x, W1, b1, W2, b2, W3, b3):
    x = jax.nn.relu(x @ W1 + b1)
    x = jax.nn.relu(x @ W2 + b2)
    return x @ W3 + b3


def _corr(x1, x2):
    x1 = x1 - jnp.mean(x1, axis=0, keepdims=True)
    x2 = x2 - jnp.mean(x2, axis=0, keepdims=True)
    s1 = jnp.sqrt(jnp.mean(x1 ** 2))
    s2 = jnp.sqrt(jnp.mean(x2 ** 2))
    return jnp.abs(jnp.mean(x1 * x2)) / (s1 * s2)


def _seg_scan(flags_start, vals, op):
    def comb(a, b):
        af, av = a
        bf, bv = b
        return (jnp.maximum(af, bf), jnp.where(bf > 0, bv, op(av, bv)))
    _, out = jax.lax.associative_scan(
        comb, (flags_start.astype(jnp.float32), vals))
    return out


def _softmax_topk_sorted(norm, dst, src, rid, topk):
    e = norm.shape[0]
    sd, neg_s, src_s, rid_s = jax.lax.sort(
        (dst, -norm, src, rid), num_keys=2, is_stable=True)
    nrm_s = -neg_s
    idx = jnp.arange(e)
    is_start = jnp.concatenate(
        [jnp.ones((1,), bool), sd[1:] != sd[:-1]])
    is_end = jnp.concatenate(
        [sd[1:] != sd[:-1], jnp.ones((1,), bool)])
    start_pos = jax.lax.associative_scan(
        jnp.maximum, jnp.where(is_start, idx, -1))
    rank = idx - start_pos
    ex = jnp.exp(nrm_s)
    cums = _seg_scan(is_start, ex, jnp.add)
    seg_total = jnp.flip(
        _seg_scan(jnp.flip(is_end), jnp.flip(cums), jnp.maximum))
    w_s = jnp.where(rank < topk, ex / seg_total, 0.0)
    return sd, src_s, rid_s, w_s


def kernel(h_id, r_id, edge_index, rel_id, ent_emb, S_w, S_b, L_w, L_b,
           rel_emb_0, rel_emb_1, neigh_w_0, neigh_w_1, pred_rel_emb,
           conv_w, conv_b, fc_w, fc_b, ent_bias,
           phi_W1, phi_b1, phi_W2, phi_b2, phi_W3, phi_b3,
           psi_W1, psi_b1, psi_W2, psi_b2, psi_W3, psi_b3):
    n_ent, h = ent_emb.shape
    topk = 15
    src = edge_index[0]
    dst = edge_index[1]

    common = ent_emb @ S_w + S_b
    private = ent_emb @ L_w + L_b

    pc0, pp0, pc1, pp1 = _edge_norm_sc(common, private, rel_emb_0, rel_emb_1,
                                       src, dst, rel_id)
    norms = {
        ('c', 0): jnp.sum(pc0, axis=1),
        ('p', 0): jnp.sum(pp0, axis=1),
        ('c', 1): jnp.sum(pc1, axis=1),
        ('p', 1): jnp.sum(pp1, axis=1),
    }

    zeros_n = jnp.zeros((n_ent, 128), jnp.float32)
    tables = {'c': common, 'p': private}
    rels = {0: rel_emb_0, 1: rel_emb_1}
    nw = {0: neigh_w_0, 1: neigh_w_1}
    cs = {}
    for l in (0, 1):
        for xk in ('c', 'p'):
            sd, src_s, rid_s, w_s = _softmax_topk_sorted(
                norms[(xk, l)], dst, src, rel_id, topk)
            neigh = _aggregate_sc(tables[xk], rels[l], src_s, sd, rid_s, w_s,
                                  zeros_n)
            cs[(xk, l)] = jnp.tanh(neigh @ nw[l])

    ent = ent_emb + cs[('c', 0)] + cs[('p', 0)] + cs[('c', 1)] + cs[('p', 1)]
    corr = _corr(
        _mlp3(cs[('c', 1)], phi_W1, phi_b1, phi_W2, phi_b2, phi_W3, phi_b3),
        _mlp3(cs[('p', 1)], psi_W1, psi_b1, psi_W2, psi_b2, psi_W3, psi_b3))

    bs = h_id.shape[0]
    kh, kw = 8, 16
    head = ent[h_id]
    rel = pred_rel_emb[r_id]
    img = jnp.concatenate([head.reshape(bs, 1, kh, kw),
                           rel.reshape(bs, 1, kh, kw)], axis=2)
    xconv = jax.lax.conv_general_dilated(
        img, conv_w, (1, 1), 'VALID', dimension_numbers=('NCHW', 'OIHW', 'NCHW'))
    xconv = jax.nn.relu(xconv + conv_b[None, :, None, None])
    xfc = jax.nn.relu(xconv.reshape(bs, -1) @ fc_w + fc_b)

    n_pad = ((n_ent + 1023) // 1024) * 1024
    entT = jnp.zeros((h, n_pad), jnp.float32).at[:, :n_ent].set(ent.T)
    bias2d = jnp.zeros((1, n_pad), jnp.float32).at[0, :n_ent].set(ent_bias)
    score = _score(xfc, entT, bias2d)[:, :n_ent]
    return (score, corr)
